# baseline (speedup 1.0000x reference)
import functools

import jax
import jax.numpy as jnp
from jax.experimental import pallas as pl
from jax.experimental.pallas import tpu as pltpu

KBLK = 216


def _matmul_bias_kernel(xt_ref, w_ref, b_ref, o_ref):
    j = pl.program_id(0)
    acc = jnp.dot(w_ref[0], xt_ref[...], preferred_element_type=jnp.float32)

    @pl.when(j == 0)
    def _():
        o_ref[...] = acc + b_ref[...]

    @pl.when(j != 0)
    def _():
        o_ref[...] = o_ref[...] + acc


@jax.jit
def kernel(x, W_last, b_last, W_dom, b_dom):
    xs = jnp.squeeze(x)
    n, k = xs.shape
    m = W_last.shape[0]
    xt = jnp.swapaxes(xs, 0, 1)
    bc = b_last.reshape(m, 1)
    w_r = W_last.reshape(m, k // KBLK, KBLK).transpose(1, 0, 2)
    grid = (k // KBLK,)
    out_t = pl.pallas_call(
        _matmul_bias_kernel,
        grid=grid,
        in_specs=[
            pl.BlockSpec((KBLK, n), lambda j: (j, 0)),
            pl.BlockSpec((1, m, KBLK), lambda j: (j, 0, 0)),
            pl.BlockSpec((m, 1), lambda j: (0, 0)),
        ],
        out_specs=pl.BlockSpec((m, n), lambda j: (0, 0)),
        out_shape=jax.ShapeDtypeStruct((m, n), jnp.float32),
        compiler_params=pltpu.CompilerParams(
            dimension_semantics=("arbitrary",),
            disable_bounds_checks=True,
        ),
    )(xt, w_r, bc)
    return jnp.swapaxes(out_t, 0, 1)


# speedup vs baseline: 1.1765x; 1.1765x over previous
import functools

import jax
import jax.numpy as jnp
from jax.experimental import pallas as pl
from jax.experimental.pallas import tpu as pltpu

BLOCK_N = 2048


def _matmul_bias_kernel(xt_ref, w_ref, b_ref, o_ref):
    o_ref[...] = (
        jnp.dot(w_ref[...], xt_ref[...], preferred_element_type=jnp.float32)
        + b_ref[...]
    )


@jax.jit
def kernel(x, W_last, b_last, W_dom, b_dom):
    xs = jnp.squeeze(x)
    n, k = xs.shape
    m = W_last.shape[0]
    xt = jnp.swapaxes(xs, 0, 1)
    bc = b_last.reshape(m, 1)
    grid = (n // BLOCK_N,)
    out_t = pl.pallas_call(
        _matmul_bias_kernel,
        grid=grid,
        in_specs=[
            pl.BlockSpec((k, BLOCK_N), lambda j: (0, j)),
            pl.BlockSpec((m, k), lambda j: (0, 0)),
            pl.BlockSpec((m, 1), lambda j: (0, 0)),
        ],
        out_specs=pl.BlockSpec((m, BLOCK_N), lambda j: (0, j)),
        out_shape=jax.ShapeDtypeStruct((m, n), jnp.float32),
        compiler_params=pltpu.CompilerParams(
            dimension_semantics=("arbitrary",),
            disable_bounds_checks=True,
        ),
    )(xt, W_last, bc)
    return jnp.swapaxes(out_t, 0, 1)
